# baseline (speedup 1.0000x reference)
import functools

import jax
import jax.numpy as jnp
from jax import lax
from jax.experimental import pallas as pl
from jax.experimental.pallas import tpu as pltpu
from jax.experimental.pallas import tpu_sc as plsc

_N = 10000
_E = 320000
_D = 128
_H = 32
_O = 64

_NC = 2
_NS = 16
_NW = _NC * _NS
_C = _O // _NW
_K = 2048
_CHUNKS = 157
_EPAD = _K * _CHUNKS


def _mlp_body(f_ref, w1_ref, b1_ref, w2_ref, b2_ref, o_ref):
    h = jnp.dot(f_ref[...], w1_ref[...], preferred_element_type=jnp.float32)
    h = jnp.maximum(h + b1_ref[...], 0.0)
    y = jnp.dot(h, w2_ref[...], preferred_element_type=jnp.float32) + b2_ref[...]
    o_ref[...] = y.T


def _mlp_t(features, W1, b1, W2, b2):
    return pl.pallas_call(
        _mlp_body,
        out_shape=jax.ShapeDtypeStruct((_O, _N), jnp.float32),
    )(features, W1, b1, W2, b2)


_mesh = plsc.VectorSubcoreMesh(core_axis_name="c", subcore_axis_name="s")


@functools.partial(
    pl.kernel,
    out_type=jax.ShapeDtypeStruct((_O, _N), jnp.float32),
    compiler_params=pltpu.CompilerParams(
        use_tc_tiling_on_sc=False, needs_layout_passes=False
    ),
    mesh=_mesh,
    scratch_types=[
        pltpu.VMEM((_C, _N), jnp.float32),
        pltpu.VMEM((_C, _N), jnp.float32),
        [pltpu.VMEM((_K,), jnp.int32)] * 2,
        [pltpu.VMEM((_K,), jnp.int32)] * 2,
        [pltpu.VMEM((_K,), jnp.float32)] * 2,
        [pltpu.SemaphoreType.DMA] * 2,
    ],
)
def _aggregate(xt_hbm, src_hbm, dst_hbm, adj_hbm, out_hbm,
               xt_v, acc_v, srcs, dsts, adjs, sems):
    cid = lax.axis_index("c")
    sid = lax.axis_index("s")
    wid = cid * _NS + sid
    colbase = wid * _C

    pltpu.sync_copy(xt_hbm.at[pl.ds(colbase, _C)], xt_v)

    @pl.loop(0, _N // 16)
    def _zero(r):
        for c in range(_C):
            acc_v[c, pl.ds(r * 16, 16)] = jnp.zeros((16,), jnp.float32)

    def _load_start(i, b):
        off = pl.ds(i * _K, _K)
        pltpu.async_copy(src_hbm.at[off], srcs[b], sems[b])
        pltpu.async_copy(dst_hbm.at[off], dsts[b], sems[b])
        pltpu.async_copy(adj_hbm.at[off], adjs[b], sems[b])

    def _load_wait(i, b):
        off = pl.ds(i * _K, _K)
        pltpu.make_async_copy(src_hbm.at[off], srcs[b], sems[b]).wait()
        pltpu.make_async_copy(dst_hbm.at[off], dsts[b], sems[b]).wait()
        pltpu.make_async_copy(adj_hbm.at[off], adjs[b], sems[b]).wait()

    def _process(b):
        @plsc.parallel_loop(0, _K // 16, unroll=8)
        def _grp(g):
            sl = pl.ds(g * 16, 16)
            sv = srcs[b][sl]
            dv = dsts[b][sl]
            av = adjs[b][sl]
            for c in range(_C):
                cv = jnp.full((16,), c, jnp.int32)
                x = plsc.load_gather(xt_v, [cv, sv])
                plsc.addupdate_scatter(acc_v, [cv, dv], x * av)

    _load_start(0, 0)

    @pl.loop(0, _CHUNKS - 1, step=2)
    def _main(i):
        _load_wait(i, 0)
        _load_start(i + 1, 1)
        _process(0)
        _load_wait(i + 1, 1)

        @pl.when(i + 2 < _CHUNKS)
        def _():
            _load_start(i + 2, 0)

        _process(1)

    _load_wait(_CHUNKS - 1, 0)
    _process(0)

    pltpu.sync_copy(acc_v, out_hbm.at[pl.ds(colbase, _C)])


def _t_body(a_ref, o_ref):
    o_ref[...] = a_ref[...].T


def _transpose_back(aggT):
    return pl.pallas_call(
        _t_body,
        out_shape=jax.ShapeDtypeStruct((_N, _O), jnp.float32),
    )(aggT)


def kernel(features, edge_index, adj_values, W1, b1, W2, b2):
    xT = _mlp_t(features, W1, b1.reshape(1, _H), W2, b2.reshape(1, _O))

    pad = _EPAD - _E
    src = jnp.pad(edge_index[1], (0, pad))
    dst = jnp.pad(edge_index[0], (0, pad))
    adj = jnp.pad(adj_values, (0, pad))

    aggT = _aggregate(xT, src, dst, adj)
    return _transpose_back(aggT)


# speedup vs baseline: 1.1460x; 1.1460x over previous
import functools

import jax
import jax.numpy as jnp
from jax import lax
from jax.experimental import pallas as pl
from jax.experimental.pallas import tpu as pltpu
from jax.experimental.pallas import tpu_sc as plsc

_N = 10000
_E = 320000
_D = 128
_H = 32
_O = 64

_NC = 2
_NS = 16
_NW = _NC * _NS
_C = _O // _NS
_K = 2048
_CHUNKS = 79
_EH = _K * _CHUNKS
_EPAD = _NC * _EH


def _mlp_body(f_ref, w1_ref, b1_ref, w2_ref, b2_ref, o_ref):
    h = jnp.dot(f_ref[...], w1_ref[...], preferred_element_type=jnp.float32)
    h = jnp.maximum(h + b1_ref[...], 0.0)
    y = jnp.dot(h, w2_ref[...], preferred_element_type=jnp.float32) + b2_ref[...]
    o_ref[...] = y.T


def _mlp_t(features, W1, b1, W2, b2):
    return pl.pallas_call(
        _mlp_body,
        out_shape=jax.ShapeDtypeStruct((_O, _N), jnp.float32),
    )(features, W1, b1, W2, b2)


_mesh = plsc.VectorSubcoreMesh(core_axis_name="c", subcore_axis_name="s")


@functools.partial(
    pl.kernel,
    out_type=jax.ShapeDtypeStruct((_NC, _O, _N), jnp.float32),
    compiler_params=pltpu.CompilerParams(
        use_tc_tiling_on_sc=False, needs_layout_passes=False
    ),
    mesh=_mesh,
    scratch_types=[
        pltpu.VMEM((_C, _N), jnp.float32),
        pltpu.VMEM((_C, _N), jnp.float32),
        [pltpu.VMEM((_K,), jnp.int32)] * 2,
        [pltpu.VMEM((_K,), jnp.int32)] * 2,
        [pltpu.VMEM((_K,), jnp.float32)] * 2,
        [pltpu.SemaphoreType.DMA] * 2,
    ],
)
def _aggregate(xt_hbm, src_hbm, dst_hbm, adj_hbm, out_hbm,
               xt_v, acc_v, srcs, dsts, adjs, sems):
    cid = lax.axis_index("c")
    sid = lax.axis_index("s")
    colbase = sid * _C

    pltpu.sync_copy(xt_hbm.at[pl.ds(colbase, _C)], xt_v)

    @pl.loop(0, _N // 16)
    def _zero(r):
        for c in range(_C):
            acc_v[c, pl.ds(r * 16, 16)] = jnp.zeros((16,), jnp.float32)

    def _load_start(i, b):
        off = pl.ds(i * _K, _K)
        pltpu.async_copy(src_hbm.at[cid, off], srcs[b], sems[b])
        pltpu.async_copy(dst_hbm.at[cid, off], dsts[b], sems[b])
        pltpu.async_copy(adj_hbm.at[cid, off], adjs[b], sems[b])

    def _load_wait(i, b):
        off = pl.ds(i * _K, _K)
        pltpu.make_async_copy(src_hbm.at[cid, off], srcs[b], sems[b]).wait()
        pltpu.make_async_copy(dst_hbm.at[cid, off], dsts[b], sems[b]).wait()
        pltpu.make_async_copy(adj_hbm.at[cid, off], adjs[b], sems[b]).wait()

    def _process(b):
        @plsc.parallel_loop(0, _K // 16, unroll=8)
        def _grp(g):
            sl = pl.ds(g * 16, 16)
            sv = srcs[b][sl]
            dv = dsts[b][sl]
            av = adjs[b][sl]
            for c in range(_C):
                cv = jnp.full((16,), c, jnp.int32)
                x = plsc.load_gather(xt_v, [cv, sv])
                plsc.addupdate_scatter(acc_v, [cv, dv], x * av)

    _load_start(0, 0)

    @pl.loop(0, _CHUNKS - 1, step=2)
    def _main(i):
        _load_wait(i, 0)
        _load_start(i + 1, 1)
        _process(0)
        _load_wait(i + 1, 1)

        @pl.when(i + 2 < _CHUNKS)
        def _():
            _load_start(i + 2, 0)

        _process(1)

    _load_wait(_CHUNKS - 1, 0)
    _process(0)

    pltpu.sync_copy(acc_v, out_hbm.at[cid, pl.ds(colbase, _C)])


def _t_body(a_ref, o_ref):
    o_ref[...] = (a_ref[0] + a_ref[1]).T


def _transpose_back(aggT):
    return pl.pallas_call(
        _t_body,
        out_shape=jax.ShapeDtypeStruct((_N, _O), jnp.float32),
    )(aggT)


def kernel(features, edge_index, adj_values, W1, b1, W2, b2):
    xT = _mlp_t(features, W1, b1.reshape(1, _H), W2, b2.reshape(1, _O))

    pad = _EPAD - _E
    src = jnp.pad(edge_index[1], (0, pad)).reshape(_NC, _EH)
    dst = jnp.pad(edge_index[0], (0, pad)).reshape(_NC, _EH)
    adj = jnp.pad(adj_values, (0, pad)).reshape(_NC, _EH)

    aggT = _aggregate(xT, src, dst, adj)
    return _transpose_back(aggT)
